# Initial kernel scaffold; baseline (speedup 1.0000x reference)
import jax
import jax.numpy as jnp
from jax.experimental import pallas as pl


def kernel(x):
    raise NotImplementedError("write your pallas kernel here")


# speedup vs baseline: 36.5197x; 36.5197x over previous
import jax
import jax.numpy as jnp
from jax.experimental import pallas as pl

_K = 9
_KBIG = 17
_LB = 90
_R = 128
_SUB = 32
_MAXF = 3.4028234663852886e38


def _knn_body(cur_ref, prev_ref, out_ref):
    t = pl.program_id(1)
    cur = cur_ref[0]
    prev = prev_ref[0]

    def _norm(v):
        n = jnp.sqrt(jnp.sum(v * v, axis=1, keepdims=True))
        return v / jnp.maximum(n, 1e-12)

    curn = _norm(cur)
    prevn = _norm(prev)
    win = jnp.concatenate([prevn, curn], axis=0)
    g = jax.lax.dot_general(
        curn, win, (((1,), (1,)), ((), ())),
        preferred_element_type=jnp.float32)
    sq_c = jnp.sum(curn * curn, axis=1, keepdims=True)
    sq_w = jnp.sum(win * win, axis=1)[None, :]
    dist = sq_c - 2.0 * g + sq_w

    base_j = (t - 1) * _R

    for s in range(_R // _SUB):
        r0 = s * _SUB
        c0 = r0 + _SUB
        d_s = dist[r0:r0 + _SUB, c0:c0 + 128]
        rr = jax.lax.broadcasted_iota(jnp.int32, (_SUB, 128), 0)
        cc = jax.lax.broadcasted_iota(jnp.int32, (_SUB, 128), 1)
        jg = base_j + c0 + cc
        ig = t * _R + r0 + rr
        diff = jg - ig
        valid = (diff <= 0) & (diff >= 1 - _LB) & (jg >= 0)
        work = jnp.where(valid, d_s, jnp.float32(_MAXF))
        jf = jg.astype(jnp.float32)

        idxs = []
        for _ in range(_KBIG):
            m = jnp.min(work, axis=1, keepdims=True)
            eq = work == m
            cand = jnp.where(eq, jf, jnp.float32(3e38))
            sel = jnp.min(cand, axis=1, keepdims=True)
            idxs.append(sel)
            work = jnp.where(jf == sel, jnp.float32(_MAXF), work)

        ig9 = t * _R + r0 + jax.lax.broadcasted_iota(jnp.int32, (_SUB, _K), 0)
        m9 = jax.lax.broadcasted_iota(jnp.int32, (_SUB, _K), 1)
        dil = jnp.concatenate([idxs[2 * m] for m in range(_K)], axis=1)
        und = jnp.concatenate(idxs[:_K], axis=1)
        head = jnp.where(m9 > ig9, idxs[0], und)
        out0 = jnp.where(ig9 >= _LB, dil, head)
        out_ref[0, 0, r0:r0 + _SUB, :] = out0.astype(jnp.int32)
        out_ref[1, 0, r0:r0 + _SUB, :] = ig9


def kernel(x):
    b, d, n, _ = x.shape
    xt = jnp.squeeze(jnp.swapaxes(x, 1, 2), -1)
    tiles = n // _R
    return pl.pallas_call(
        _knn_body,
        grid=(b, tiles),
        in_specs=[
            pl.BlockSpec((1, _R, d), lambda bi, ti: (bi, ti, 0)),
            pl.BlockSpec((1, _R, d), lambda bi, ti: (bi, jnp.maximum(ti - 1, 0), 0)),
        ],
        out_specs=pl.BlockSpec((2, 1, _R, _K), lambda bi, ti: (0, bi, ti, 0)),
        out_shape=jax.ShapeDtypeStruct((2, b, n, _K), jnp.int32),
    )(xt, xt)


# speedup vs baseline: 61.2725x; 1.6778x over previous
import jax
import jax.numpy as jnp
from jax.experimental import pallas as pl

_K = 9
_KBIG = 17
_LB = 90
_R = 128
_SUB = 8
_NSUB = _R // _SUB
_MAXF = 3.4028234663852886e38


def _knn_body(cur_ref, prev_ref, out_ref):
    t = pl.program_id(1)
    cur = cur_ref[0]
    prev = prev_ref[0]

    def _norm(v):
        n = jnp.sqrt(jnp.sum(v * v, axis=1, keepdims=True))
        return v / jnp.maximum(n, 1e-12)

    curn = _norm(cur)
    prevn = _norm(prev)
    win = jnp.concatenate([prevn, curn], axis=0)
    g = jax.lax.dot_general(
        curn, win, (((1,), (1,)), ((), ())),
        preferred_element_type=jnp.float32)
    sq_c = jnp.sum(curn * curn, axis=1, keepdims=True)
    sq_w = jnp.sum(win * win, axis=1)[None, :]
    dist = sq_c - 2.0 * g + sq_w

    base_j = (t - 1) * _R

    rr = jax.lax.broadcasted_iota(jnp.int32, (_R, 2 * _R), 0)
    cc = jax.lax.broadcasted_iota(jnp.int32, (_R, 2 * _R), 1)
    diff = cc - rr - _R
    valid = (diff <= 0) & (diff >= 1 - _LB) & (base_j + cc >= 0)
    ki = jax.lax.bitcast_convert_type(dist + 1.0, jnp.int32)
    ki = (ki & jnp.int32(-128)) | (cc & 127)
    key = jnp.where(valid, jax.lax.bitcast_convert_type(ki, jnp.float32),
                    jnp.float32(_MAXF))

    works = []
    acc_d = []
    acc_u = []
    for s in range(_NSUB):
        c0 = s * _SUB + _SUB
        works.append(key[s * _SUB:(s + 1) * _SUB, c0:c0 + _R])
        acc_d.append(jnp.full((_SUB, _R), _MAXF, jnp.float32))
        acc_u.append(jnp.full((_SUB, _R), _MAXF, jnp.float32))

    lane = jax.lax.broadcasted_iota(jnp.int32, (_SUB, _R), 1)
    for k in range(_KBIG):
        lane_d = lane == (k // 2)
        lane_u = lane == k
        for s in range(_NSUB):
            m = jnp.min(works[s], axis=1, keepdims=True)
            mb = jnp.broadcast_to(m, (_SUB, _R))
            works[s] = jnp.where(works[s] == mb, jnp.float32(_MAXF), works[s])
            if k % 2 == 0:
                acc_d[s] = jnp.where(lane_d, mb, acc_d[s])
            if k < _K:
                acc_u[s] = jnp.where(lane_u, mb, acc_u[s])

    m9 = jax.lax.broadcasted_iota(jnp.int32, (_SUB, _K), 1)
    for s in range(_NSUB):
        r0 = s * _SUB
        c0 = r0 + _SUB
        und = acc_u[s][:, :_K]
        dil = acc_d[s][:, :_K]
        fill0 = jnp.broadcast_to(acc_u[s][:, 0:1], (_SUB, _K))
        ig9 = t * _R + r0 + jax.lax.broadcasted_iota(jnp.int32, (_SUB, _K), 0)
        head = jnp.where(m9 > ig9, fill0, und)
        out_keys = jnp.where(ig9 >= _LB, dil, head)
        oki = jax.lax.bitcast_convert_type(out_keys, jnp.int32)
        jg = base_j + c0 + ((oki - c0) & 127)
        out_ref[0, 0, r0:r0 + _SUB, :] = jg
        out_ref[1, 0, r0:r0 + _SUB, :] = ig9


def kernel(x):
    b, d, n, _ = x.shape
    xt = jnp.squeeze(jnp.swapaxes(x, 1, 2), -1)
    tiles = n // _R
    return pl.pallas_call(
        _knn_body,
        grid=(b, tiles),
        in_specs=[
            pl.BlockSpec((1, _R, d), lambda bi, ti: (bi, ti, 0)),
            pl.BlockSpec((1, _R, d), lambda bi, ti: (bi, jnp.maximum(ti - 1, 0), 0)),
        ],
        out_specs=pl.BlockSpec((2, 1, _R, _K), lambda bi, ti: (0, bi, ti, 0)),
        out_shape=jax.ShapeDtypeStruct((2, b, n, _K), jnp.int32),
    )(xt, xt)


# speedup vs baseline: 91.0574x; 1.4861x over previous
import jax
import jax.numpy as jnp
from jax.experimental import pallas as pl
from jax.experimental.pallas import tpu as pltpu

_K = 9
_KBIG = 17
_LB = 90
_R = 256
_H = 128
_SUB = 8
_NSUB = _R // _SUB
_MAXF = 3.4028234663852886e38


def _knn_body(cur_ref, prev_ref, out_ref, dsc_ref, usc_ref):
    t = pl.program_id(1)
    cur = cur_ref[0]
    halo = prev_ref[0][_R - _H:]

    def _norm(v):
        n2 = jnp.sum(v * v, axis=1, keepdims=True)
        return v * jax.lax.rsqrt(jnp.maximum(n2, 1e-24))

    curn = _norm(cur)
    halon = _norm(halo)
    win = jnp.concatenate([halon, curn], axis=0)
    g = jax.lax.dot_general(
        curn, win, (((1,), (1,)), ((), ())),
        preferred_element_type=jnp.float32)
    sq_c = jnp.sum(curn * curn, axis=1, keepdims=True)
    sq_w = jnp.sum(win * win, axis=1)[None, :]
    dist = sq_c - 2.0 * g + sq_w

    base_j = t * _R - _H

    rr8 = jax.lax.broadcasted_iota(jnp.int32, (_SUB, 128), 0)
    lane = jax.lax.broadcasted_iota(jnp.int32, (_SUB, 128), 1)
    diffc = lane - rr8 - (_H - _SUB)
    band_ok = (diffc <= 0) & (diffc >= 1 - _LB)

    works = []
    for s in range(_NSUB):
        r0 = s * _SUB
        c0 = r0 + _SUB
        d_s = dist[r0:r0 + _SUB, c0:c0 + 128]
        ki = jax.lax.bitcast_convert_type(d_s + 1.0, jnp.int32)
        ki = (ki & jnp.int32(-128)) | ((lane + c0) & 127)
        ok = band_ok & (base_j + c0 + lane >= 0)
        works.append(jnp.where(
            ok, jax.lax.bitcast_convert_type(ki, jnp.float32),
            jnp.float32(_MAXF)))

    for k in range(_KBIG):
        for s in range(_NSUB):
            r0 = s * _SUB
            m = jnp.min(works[s], axis=1, keepdims=True)
            mb = jnp.broadcast_to(m, (_SUB, 128))
            works[s] = jnp.where(works[s] == mb, jnp.float32(_MAXF), works[s])
            if k % 2 == 0:
                dsc_ref[r0:r0 + _SUB, k // 2:k // 2 + 1] = m
            if k < _K:
                usc_ref[r0:r0 + _SUB, k:k + 1] = m

    dil = dsc_ref[:, :_K]
    und = usc_ref[:, :_K]
    fill0 = jnp.broadcast_to(usc_ref[:, 0:1], (_R, _K))
    r9 = jax.lax.broadcasted_iota(jnp.int32, (_R, _K), 0)
    m9 = jax.lax.broadcasted_iota(jnp.int32, (_R, _K), 1)
    ig9 = t * _R + r9
    head = jnp.where(m9 > ig9, fill0, und)
    out_keys = jnp.where(ig9 >= _LB, dil, head)
    oki = jax.lax.bitcast_convert_type(out_keys, jnp.int32)
    c0v = (r9 & jnp.int32(-_SUB)) + _SUB
    jg = base_j + c0v + ((oki - c0v) & 127)
    out_ref[0, 0, :, :] = jg
    out_ref[1, 0, :, :] = ig9


def kernel(x):
    b, d, n, _ = x.shape
    xt = jnp.squeeze(jnp.swapaxes(x, 1, 2), -1)
    tiles = n // _R
    return pl.pallas_call(
        _knn_body,
        grid=(b, tiles),
        in_specs=[
            pl.BlockSpec((1, _R, d), lambda bi, ti: (bi, ti, 0)),
            pl.BlockSpec((1, _R, d), lambda bi, ti: (bi, jnp.maximum(ti - 1, 0), 0)),
        ],
        out_specs=pl.BlockSpec((2, 1, _R, _K), lambda bi, ti: (0, bi, ti, 0)),
        out_shape=jax.ShapeDtypeStruct((2, b, n, _K), jnp.int32),
        scratch_shapes=[
            pltpu.VMEM((_R, 16), jnp.float32),
            pltpu.VMEM((_R, 16), jnp.float32),
        ],
    )(xt, xt)


# speedup vs baseline: 198.5145x; 2.1801x over previous
import jax
import jax.numpy as jnp
from jax.experimental import pallas as pl
from jax.experimental.pallas import tpu as pltpu

_K = 9
_KBIG = 17
_LB = 90
_R = 256
_H = 128
_W = _R + _H
_MAXF = 3.4028234663852886e38


def _knn_body(cur_ref, prev_ref, out_ref, dsc_ref, usc_ref):
    t = pl.program_id(1)
    cur = cur_ref[0]
    halo = prev_ref[0][_R - _H:]

    def _norm(v):
        n2 = jnp.sum(v * v, axis=1, keepdims=True)
        return v * jax.lax.rsqrt(jnp.maximum(n2, 1e-24))

    curn = _norm(cur)
    halon = _norm(halo)
    win = jnp.concatenate([halon, curn], axis=0)
    g = jax.lax.dot_general(
        win, curn, (((1,), (1,)), ((), ())),
        preferred_element_type=jnp.float32)
    sq_w = jnp.sum(win * win, axis=1, keepdims=True)
    sqc = jax.lax.dot_general(
        jnp.ones((8, 64), jnp.float32), curn * curn,
        (((1,), (1,)), ((), ())),
        preferred_element_type=jnp.float32)
    g2 = (sq_w + 1.0) - 2.0 * g + jnp.broadcast_to(sqc[0:1], (_W, _R))

    keep = {256: 384, 128: 256, 64: 192, 32: 160, 16: 144, 8: 144,
            4: 144, 2: 144, 1: 144}
    for b in (256, 128, 64, 32, 16, 8, 4, 2, 1):
        n = g2.shape[0]
        rolled = pltpu.roll(g2, n - b, 0)
        rp1 = jax.lax.broadcasted_iota(jnp.int32, (n, _R), 1) + 1
        g2 = jnp.where((rp1 & b) != 0, rolled, g2)
        if keep[b] < n:
            g2 = g2[:keep[b]]
    tile = g2[32:128]

    s96 = jax.lax.broadcasted_iota(jnp.int32, (96, _R), 0)
    lane = jax.lax.broadcasted_iota(jnp.int32, (96, _R), 1)
    d = 95 - s96
    ig = t * _R + lane
    valid = (d <= _LB - 1) & (d <= ig)
    ki = (jax.lax.bitcast_convert_type(tile, jnp.int32) & jnp.int32(-128)) | s96
    works = jnp.where(valid, jax.lax.bitcast_convert_type(ki, jnp.float32),
                      jnp.float32(_MAXF))

    for k in range(_KBIG):
        m = jnp.min(works, axis=0, keepdims=True)
        mb = jnp.broadcast_to(m, (96, _R))
        works = jnp.where(works == mb, jnp.float32(_MAXF), works)
        if k % 2 == 0:
            dsc_ref[k // 2:k // 2 + 1, :] = m
        if k < _K:
            usc_ref[k:k + 1, :] = m

    dil = dsc_ref[0:_K, :]
    und = usc_ref[0:_K, :]
    fill0 = jnp.broadcast_to(usc_ref[0:1, :], (_K, _R))
    m9 = jax.lax.broadcasted_iota(jnp.int32, (_K, _R), 0)
    ig9 = t * _R + jax.lax.broadcasted_iota(jnp.int32, (_K, _R), 1)
    head = jnp.where(m9 > ig9, fill0, und)
    out_keys = jnp.where(ig9 >= _LB, dil, head)
    oki = jax.lax.bitcast_convert_type(out_keys, jnp.int32)
    jg = ig9 - 95 + (oki & 127)
    out_ref[0, 0, :, :] = jg
    out_ref[0, 1, :, :] = ig9


def kernel(x):
    b, dd, n, _ = x.shape
    xt = jnp.squeeze(jnp.swapaxes(x, 1, 2), -1)
    tiles = n // _R
    raw = pl.pallas_call(
        _knn_body,
        grid=(b, tiles),
        in_specs=[
            pl.BlockSpec((1, _R, dd), lambda bi, ti: (bi, ti, 0)),
            pl.BlockSpec((1, _R, dd), lambda bi, ti: (bi, jnp.maximum(ti - 1, 0), 0)),
        ],
        out_specs=pl.BlockSpec((1, 2, _K, _R), lambda bi, ti: (bi, 0, 0, ti)),
        out_shape=jax.ShapeDtypeStruct((b, 2, _K, n), jnp.int32),
        scratch_shapes=[
            pltpu.VMEM((16, _R), jnp.float32),
            pltpu.VMEM((16, _R), jnp.float32),
        ],
    )(xt, xt)
    return jnp.transpose(raw, (1, 0, 3, 2))


# speedup vs baseline: 216.4602x; 1.0904x over previous
import jax
import jax.numpy as jnp
from jax.experimental import pallas as pl
from jax.experimental.pallas import tpu as pltpu

_K = 9
_KBIG = 17
_LB = 90
_R = 256
_H = 128
_W = _R + _H
_MAXF = 3.4028234663852886e38


def _knn_body(cur_ref, prev_ref, out_ref, dsc_ref, usc_ref):
    t = pl.program_id(1)
    cur = cur_ref[0]
    halo = prev_ref[0][:, _R - _H:]

    def _norm(v):
        n2 = jnp.sum(v * v, axis=0, keepdims=True)
        return v * jax.lax.rsqrt(jnp.maximum(n2, 1e-24))

    curn = _norm(cur)
    halon = _norm(halo)
    win = jnp.concatenate([halon, curn], axis=1)
    g = jax.lax.dot_general(
        win, curn, (((0,), (0,)), ((), ())),
        preferred_element_type=jnp.float32)
    sqc = jnp.sum(curn * curn, axis=0, keepdims=True)
    sq_w = jax.lax.dot_general(
        win * win, jnp.ones((64, 8), jnp.float32), (((0,), (0,)), ((), ())),
        preferred_element_type=jnp.float32)[:, 0:1]
    g2 = (sq_w + 1.0) - 2.0 * g + jnp.broadcast_to(sqc, (_W, _R))

    keep = {256: 384, 128: 256, 64: 192, 32: 160, 16: 144, 8: 144,
            4: 144, 2: 144, 1: 144}
    for b in (256, 128, 64, 32, 16, 8, 4, 2, 1):
        n = g2.shape[0]
        rolled = pltpu.roll(g2, n - b, 0)
        rp1 = jax.lax.broadcasted_iota(jnp.int32, (n, _R), 1) + 1
        g2 = jnp.where((rp1 & b) != 0, rolled, g2)
        if keep[b] < n:
            g2 = g2[:keep[b]]
    tile = g2[32:128]

    s96 = jax.lax.broadcasted_iota(jnp.int32, (96, _R), 0)
    lane = jax.lax.broadcasted_iota(jnp.int32, (96, _R), 1)
    d = 95 - s96
    ig = t * _R + lane
    valid = (d <= _LB - 1) & (d <= ig)
    ki = (jax.lax.bitcast_convert_type(tile, jnp.int32) & jnp.int32(-128)) | s96
    works = jnp.where(valid, jax.lax.bitcast_convert_type(ki, jnp.float32),
                      jnp.float32(_MAXF))

    for k in range(_KBIG):
        m = jnp.min(works, axis=0, keepdims=True)
        mb = jnp.broadcast_to(m, (96, _R))
        works = jnp.where(works == mb, jnp.float32(_MAXF), works)
        if k % 2 == 0:
            dsc_ref[k // 2:k // 2 + 1, :] = m
        if k < _K:
            usc_ref[k:k + 1, :] = m

    dil = dsc_ref[0:_K, :]
    und = usc_ref[0:_K, :]
    fill0 = jnp.broadcast_to(usc_ref[0:1, :], (_K, _R))
    m9 = jax.lax.broadcasted_iota(jnp.int32, (_K, _R), 0)
    ig9 = t * _R + jax.lax.broadcasted_iota(jnp.int32, (_K, _R), 1)
    head = jnp.where(m9 > ig9, fill0, und)
    out_keys = jnp.where(ig9 >= _LB, dil, head)
    oki = jax.lax.bitcast_convert_type(out_keys, jnp.int32)
    jg = ig9 - 95 + (oki & 127)
    out_ref[0, 0, :, :] = jg
    out_ref[0, 1, :, :] = ig9


def kernel(x):
    b, dd, n, _ = x.shape
    x3 = x[..., 0]
    tiles = n // _R
    raw = pl.pallas_call(
        _knn_body,
        grid=(b, tiles),
        in_specs=[
            pl.BlockSpec((1, dd, _R), lambda bi, ti: (bi, 0, ti)),
            pl.BlockSpec((1, dd, _R), lambda bi, ti: (bi, 0, jnp.maximum(ti - 1, 0))),
        ],
        out_specs=pl.BlockSpec((1, 2, _K, _R), lambda bi, ti: (bi, 0, 0, ti)),
        out_shape=jax.ShapeDtypeStruct((b, 2, _K, n), jnp.int32),
        scratch_shapes=[
            pltpu.VMEM((16, _R), jnp.float32),
            pltpu.VMEM((16, _R), jnp.float32),
        ],
    )(x3, x3)
    return jnp.transpose(raw, (1, 0, 3, 2))


# speedup vs baseline: 227.3330x; 1.0502x over previous
import jax
import jax.numpy as jnp
from jax.experimental import pallas as pl
from jax.experimental.pallas import tpu as pltpu

_K = 9
_KBIG = 17
_LB = 90
_R = 256
_H = 128
_W = _R + _H
_MAXF = 3.4028234663852886e38


def _knn_body(cur_ref, prev_ref, out_ref, dsc_ref, usc_ref):
    t = pl.program_id(1)
    cur = cur_ref[0]
    halo = prev_ref[0][:, _R - _H:]

    def _inv(v):
        n2 = jnp.sum(v * v, axis=0, keepdims=True)
        return jax.lax.rsqrt(jnp.maximum(n2, 1e-24))

    win = jnp.concatenate([halo * _inv(halo), cur * _inv(cur)], axis=1)
    g2 = jax.lax.dot_general(
        win, cur, (((0,), (0,)), ((), ())),
        preferred_element_type=jnp.float32)
    invc = jax.lax.rsqrt(
        jnp.maximum(jnp.sum(cur * cur, axis=0, keepdims=True), 1e-24))

    keep = {256: 384, 128: 256, 64: 192, 32: 160, 16: 144, 8: 144,
            4: 144, 2: 144, 1: 144}
    for b in (256, 128, 64, 32, 16, 8, 4, 2, 1):
        n = g2.shape[0]
        rolled = pltpu.roll(g2, n - b, 0)
        rp1 = jax.lax.broadcasted_iota(jnp.int32, (n, _R), 1) + 1
        g2 = jnp.where((rp1 & b) != 0, rolled, g2)
        if keep[b] < n:
            g2 = g2[:keep[b]]
    tile = 1.5 - g2[32:128] * jnp.broadcast_to(invc, (96, _R))

    s96 = jax.lax.broadcasted_iota(jnp.int32, (96, _R), 0)
    lane = jax.lax.broadcasted_iota(jnp.int32, (96, _R), 1)
    d = 95 - s96
    ig = t * _R + lane
    valid = (d <= _LB - 1) & (d <= ig)
    ki = (jax.lax.bitcast_convert_type(tile, jnp.int32) & jnp.int32(-128)) | s96
    works = jnp.where(valid, jax.lax.bitcast_convert_type(ki, jnp.float32),
                      jnp.float32(_MAXF))

    halves = [works[:, :128], works[:, 128:]]
    for k in range(_KBIG):
        for h in (0, 1):
            m = jnp.min(halves[h], axis=0, keepdims=True)
            mb = jnp.broadcast_to(m, (96, 128))
            halves[h] = jnp.where(halves[h] == mb, jnp.float32(_MAXF),
                                  halves[h])
            if k % 2 == 0:
                dsc_ref[k // 2:k // 2 + 1, h * 128:h * 128 + 128] = m
            if k < _K:
                usc_ref[k:k + 1, h * 128:h * 128 + 128] = m

    dil = dsc_ref[0:_K, :]
    und = usc_ref[0:_K, :]
    fill0 = jnp.broadcast_to(usc_ref[0:1, :], (_K, _R))
    m9 = jax.lax.broadcasted_iota(jnp.int32, (_K, _R), 0)
    ig9 = t * _R + jax.lax.broadcasted_iota(jnp.int32, (_K, _R), 1)
    head = jnp.where(m9 > ig9, fill0, und)
    out_keys = jnp.where(ig9 >= _LB, dil, head)
    oki = jax.lax.bitcast_convert_type(out_keys, jnp.int32)
    jg = ig9 - 95 + (oki & 127)
    out_ref[0, 0, :, :] = jg
    out_ref[0, 1, :, :] = ig9


def kernel(x):
    b, dd, n, _ = x.shape
    x3 = x[..., 0]
    tiles = n // _R
    raw = pl.pallas_call(
        _knn_body,
        grid=(b, tiles),
        in_specs=[
            pl.BlockSpec((1, dd, _R), lambda bi, ti: (bi, 0, ti)),
            pl.BlockSpec((1, dd, _R), lambda bi, ti: (bi, 0, jnp.maximum(ti - 1, 0))),
        ],
        out_specs=pl.BlockSpec((1, 2, _K, _R), lambda bi, ti: (bi, 0, 0, ti)),
        out_shape=jax.ShapeDtypeStruct((b, 2, _K, n), jnp.int32),
        scratch_shapes=[
            pltpu.VMEM((16, _R), jnp.float32),
            pltpu.VMEM((16, _R), jnp.float32),
        ],
    )(x3, x3)
    return jnp.transpose(raw, (1, 0, 3, 2))


# speedup vs baseline: 238.0043x; 1.0469x over previous
import jax
import jax.numpy as jnp
from jax.experimental import pallas as pl
from jax.experimental.pallas import tpu as pltpu

_K = 9
_KBIG = 17
_LB = 90
_R = 256
_H = 128
_W = _R + _H
_MAXF = 3.4028234663852886e38


def _knn_body(cur_ref, prev_ref, out_ref, ksc_ref, dsc_ref, usc_ref):
    t = pl.program_id(1)
    cur = cur_ref[0]
    halo = prev_ref[0][:, _R - _H:]

    def _inv(v):
        n2 = jnp.sum(v * v, axis=0, keepdims=True)
        return jax.lax.rsqrt(jnp.maximum(n2, 1e-24))

    win = jnp.concatenate([halo * _inv(halo), cur * _inv(cur)], axis=1)
    g2 = jax.lax.dot_general(
        win, cur, (((0,), (0,)), ((), ())),
        preferred_element_type=jnp.float32)
    invc = jax.lax.rsqrt(
        jnp.maximum(jnp.sum(cur * cur, axis=0, keepdims=True), 1e-24))

    keep = {256: 384, 128: 256, 64: 192, 32: 160, 16: 144, 8: 144,
            4: 144, 2: 144, 1: 144}
    for b in (256, 128, 64, 32, 16, 8, 4, 2, 1):
        n = g2.shape[0]
        rolled = pltpu.roll(g2, n - b, 0)
        rp1 = jax.lax.broadcasted_iota(jnp.int32, (n, _R), 1) + 1
        g2 = jnp.where((rp1 & b) != 0, rolled, g2)
        if keep[b] < n:
            g2 = g2[:keep[b]]
    tile = 1.5 - g2[32:128] * jnp.broadcast_to(invc, (96, _R))

    s96 = jax.lax.broadcasted_iota(jnp.int32, (96, _R), 0)
    lane = jax.lax.broadcasted_iota(jnp.int32, (96, _R), 1)
    d = 95 - s96
    ig = t * _R + lane
    valid = (d <= _LB - 1) & (d <= ig)
    ki = (jax.lax.bitcast_convert_type(tile, jnp.int32) & jnp.int32(-128)) | s96
    works = jnp.where(valid, jax.lax.bitcast_convert_type(ki, jnp.float32),
                      jnp.float32(_MAXF))

    tprev = t - 1
    wprev = ksc_ref[pl.ds(((t + 1) % 2) * 96, 96), :]

    halves = [wprev[:, :128], wprev[:, 128:]]
    for k in range(_KBIG):
        for h in (0, 1):
            m = jnp.min(halves[h], axis=0, keepdims=True)
            mb = jnp.broadcast_to(m, (96, 128))
            halves[h] = jnp.where(halves[h] == mb, jnp.float32(_MAXF),
                                  halves[h])
            if k % 2 == 0:
                dsc_ref[k // 2:k // 2 + 1, h * 128:h * 128 + 128] = m
            if k < _K:
                usc_ref[k:k + 1, h * 128:h * 128 + 128] = m

    dil = dsc_ref[0:_K, :]
    und = usc_ref[0:_K, :]
    fill0 = jnp.broadcast_to(usc_ref[0:1, :], (_K, _R))
    m9 = jax.lax.broadcasted_iota(jnp.int32, (_K, _R), 0)
    ig9 = tprev * _R + jax.lax.broadcasted_iota(jnp.int32, (_K, _R), 1)
    head = jnp.where(m9 > ig9, fill0, und)
    out_keys = jnp.where(ig9 >= _LB, dil, head)
    oki = jax.lax.bitcast_convert_type(out_keys, jnp.int32)
    jg = ig9 - 95 + (oki & 127)
    out_ref[0, 0, :, :] = jg
    out_ref[0, 1, :, :] = ig9
    ksc_ref[pl.ds((t % 2) * 96, 96), :] = works


def kernel(x):
    b, dd, n, _ = x.shape
    x3 = x[..., 0]
    tiles = n // _R
    raw = pl.pallas_call(
        _knn_body,
        grid=(b, tiles + 1),
        in_specs=[
            pl.BlockSpec(
                (1, dd, _R),
                lambda bi, ti: (bi, 0, jnp.minimum(ti, tiles - 1))),
            pl.BlockSpec(
                (1, dd, _R),
                lambda bi, ti: (
                    bi, 0, jnp.maximum(jnp.minimum(ti, tiles - 1) - 1, 0))),
        ],
        out_specs=pl.BlockSpec(
            (1, 2, _K, _R), lambda bi, ti: (bi, 0, 0, jnp.maximum(ti - 1, 0))),
        out_shape=jax.ShapeDtypeStruct((b, 2, _K, n), jnp.int32),
        scratch_shapes=[
            pltpu.VMEM((192, _R), jnp.float32),
            pltpu.VMEM((16, _R), jnp.float32),
            pltpu.VMEM((16, _R), jnp.float32),
        ],
    )(x3, x3)
    return jnp.transpose(raw, (1, 0, 3, 2))


# speedup vs baseline: 304.7424x; 1.2804x over previous
import jax
import jax.numpy as jnp
from jax.experimental import pallas as pl
from jax.experimental.pallas import tpu as pltpu

_K = 9
_KBIG = 17
_LB = 90
_R = 256
_S = 512
_H = 128
_MAXF = 3.4028234663852886e38


def _produce(cur, halo, teff):
    def _inv(v):
        n2 = jnp.sum(v * v, axis=0, keepdims=True)
        return jax.lax.rsqrt(jnp.maximum(n2, 1e-24))

    win = jnp.concatenate([halo * _inv(halo), cur * _inv(cur)], axis=1)
    g2 = jax.lax.dot_general(
        win, cur, (((0,), (0,)), ((), ())),
        preferred_element_type=jnp.float32)
    invc = jax.lax.rsqrt(
        jnp.maximum(jnp.sum(cur * cur, axis=0, keepdims=True), 1e-24))

    keep = {256: 384, 128: 256, 64: 192, 32: 160, 16: 144, 8: 144,
            4: 144, 2: 144, 1: 144}
    for b in (256, 128, 64, 32, 16, 8, 4, 2, 1):
        n = g2.shape[0]
        rolled = pltpu.roll(g2, n - b, 0)
        rp1 = jax.lax.broadcasted_iota(jnp.int32, (n, _R), 1) + 1
        g2 = jnp.where((rp1 & b) != 0, rolled, g2)
        if keep[b] < n:
            g2 = g2[:keep[b]]
    tile = 1.5 - g2[32:128] * jnp.broadcast_to(invc, (96, _R))

    s96 = jax.lax.broadcasted_iota(jnp.int32, (96, _R), 0)
    lane = jax.lax.broadcasted_iota(jnp.int32, (96, _R), 1)
    d = 95 - s96
    ig = teff * _R + lane
    valid = (d <= _LB - 1) & (d <= ig)
    ki = (jax.lax.bitcast_convert_type(tile, jnp.int32) & jnp.int32(-128)) | s96
    return jnp.where(valid, jax.lax.bitcast_convert_type(ki, jnp.float32),
                     jnp.float32(_MAXF))


def _knn_body(cur_ref, prev_ref, out_ref, ksc_ref, dsc_ref, usc_ref):
    t = pl.program_id(1)
    blk = cur_ref[0]
    pblk = prev_ref[0]

    works = [
        _produce(blk[:, :_R], pblk[:, _S - _H:], 2 * t),
        _produce(blk[:, _R:], blk[:, _R - _H:_R], 2 * t + 1),
    ]

    par_w = (t % 2) * 192
    par_r = ((t + 1) % 2) * 192
    chains = []
    for u in (0, 1):
        wprev = ksc_ref[pl.ds(par_r + u * 96, 96), :]
        chains.append(wprev[:, :128])
        chains.append(wprev[:, 128:])

    for k in range(_KBIG):
        for c in range(4):
            m = jnp.min(chains[c], axis=0, keepdims=True)
            mb = jnp.broadcast_to(m, (96, 128))
            chains[c] = jnp.where(chains[c] == mb, jnp.float32(_MAXF),
                                  chains[c])
            if k % 2 == 0:
                dsc_ref[k // 2:k // 2 + 1, c * 128:c * 128 + 128] = m
            if k < _K:
                usc_ref[k:k + 1, c * 128:c * 128 + 128] = m

    dil = dsc_ref[0:_K, :]
    und = usc_ref[0:_K, :]
    fill0 = jnp.broadcast_to(usc_ref[0:1, :], (_K, _S))
    m9 = jax.lax.broadcasted_iota(jnp.int32, (_K, _S), 0)
    ig9 = (t - 1) * _S + jax.lax.broadcasted_iota(jnp.int32, (_K, _S), 1)
    head = jnp.where(m9 > ig9, fill0, und)
    out_keys = jnp.where(ig9 >= _LB, dil, head)
    oki = jax.lax.bitcast_convert_type(out_keys, jnp.int32)
    jg = ig9 - 95 + (oki & 127)
    out_ref[0, 0, :, :] = jg
    out_ref[0, 1, :, :] = ig9

    ksc_ref[pl.ds(par_w, 96), :] = works[0]
    ksc_ref[pl.ds(par_w + 96, 96), :] = works[1]


def kernel(x):
    b, dd, n, _ = x.shape
    x3 = x[..., 0]
    steps = n // _S
    raw = pl.pallas_call(
        _knn_body,
        grid=(b, steps + 1),
        in_specs=[
            pl.BlockSpec(
                (1, dd, _S),
                lambda bi, ti: (bi, 0, jnp.minimum(ti, steps - 1))),
            pl.BlockSpec(
                (1, dd, _S),
                lambda bi, ti: (
                    bi, 0, jnp.maximum(jnp.minimum(ti, steps - 1) - 1, 0))),
        ],
        out_specs=pl.BlockSpec(
            (1, 2, _K, _S), lambda bi, ti: (bi, 0, 0, jnp.maximum(ti - 1, 0))),
        out_shape=jax.ShapeDtypeStruct((b, 2, _K, n), jnp.int32),
        scratch_shapes=[
            pltpu.VMEM((384, _R), jnp.float32),
            pltpu.VMEM((16, _S), jnp.float32),
            pltpu.VMEM((16, _S), jnp.float32),
        ],
    )(x3, x3)
    return jnp.transpose(raw, (1, 0, 3, 2))
